# baseline (speedup 1.0000x reference)
import jax
import jax.numpy as jnp
from jax import lax
from jax.experimental import pallas as pl
from jax.experimental.pallas import tpu as pltpu
from jax.experimental.pallas import tpu_sc as plsc

B = 1024
L = 200
EMB = 16
NC = 2
NS = 16
NW = NC * NS
ROWS_PER_W = B // NW
GATHER_PER_W = ROWS_PER_W * L
CHUNK = 128
NCHUNK = GATHER_PER_W // CHUNK
NBUF = 4
TROWS = 125000
NCHUNK_PAD = 56


def _sc_body(seq_hbm, items_hbm, t128_hbm, u_hbm, v_hbm,
             idx_v, qbuf, lanebuf, bufs, usum_v, itm_v,
             vbuf, vrows_v, sem, sem2):
    wid = lax.axis_index("s") * NC + lax.axis_index("c")
    base = wid * ROWS_PER_W
    iota = lax.iota(jnp.int32, 16)

    pltpu.sync_copy(seq_hbm.at[pl.ds(wid * NCHUNK_PAD, NCHUNK_PAD)], idx_v)

    def xform(i, c):
        for o8 in range(8):
            sl = idx_v[i, pl.ds(o8 * 16, 16)]
            qbuf[i, pl.ds(o8 * 16, 16)] = lax.shift_right_logical(sl, 3)
            lanebuf[i, pl.ds(o8 * 16, 16)] = lax.shift_left(
                lax.bitwise_and(sl, 7), 4)
        return c
    lax.fori_loop(0, NCHUNK, xform, 0)

    zero16 = jnp.zeros((EMB,), jnp.float32)
    def zinit(r, c):
        usum_v[r] = zero16
        return c
    lax.fori_loop(0, ROWS_PER_W, zinit, 0)

    for k0 in range(NBUF - 1):
        pltpu.async_copy(t128_hbm.at[qbuf.at[k0]], bufs.at[k0], sem)

    pltpu.sync_copy(items_hbm.at[wid], itm_v)
    vcps = []
    for jb in range(2):
        sl = itm_v[pl.ds(jb * 16, 16)]
        qv = lax.shift_right_logical(sl, 3)
        vcps.append(pltpu.async_copy(
            t128_hbm.at[qv], vbuf.at[pl.ds(jb * 16, 16)], sem2))
    for cp in vcps:
        cp.wait()
    for jb in range(2):
        sl = itm_v[pl.ds(jb * 16, 16)]
        off16 = lax.shift_left(lax.bitwise_and(sl, 7), 4)
        rows16 = iota + jb * 16
        for c in range(16):
            vals = plsc.load_gather(vbuf, [rows16, off16 + c])
            plsc.store_scatter(
                vrows_v, [rows16, jnp.full((16,), c, jnp.int32)], vals)

    def chunk_step(k, c):
        pltpu.make_async_copy(t128_hbm.at[qbuf.at[0]],
                              bufs.at[0], sem).wait()
        slot = lax.rem(k, NBUF)
        slotv = jnp.full((16,), slot, jnp.int32)

        def block_step(b, c2):
            jloc = b * 16
            for k2 in range(16):
                jg = k * CHUNK + jloc + k2
                r = lax.div(jg, L)
                off = plsc.load_gather(
                    lanebuf, [jnp.full((16,), k, jnp.int32),
                              jnp.full((16,), jloc + k2, jnp.int32)])
                val = plsc.load_gather(
                    bufs, [slotv, jnp.full((16,), jloc + k2, jnp.int32),
                           off + iota])
                plsc.addupdate(usum_v.at[r], val)
            return c2
        lax.fori_loop(0, CHUNK // 16, block_step, 0)

        @pl.when(k + NBUF - 1 < NCHUNK)
        def _():
            pltpu.async_copy(t128_hbm.at[qbuf.at[k + NBUF - 1]],
                             bufs.at[lax.rem(k + NBUF - 1, NBUF)], sem)
        return c
    lax.fori_loop(0, NCHUNK, chunk_step, 0)

    pltpu.sync_copy(usum_v, u_hbm.at[pl.ds(base, ROWS_PER_W)])
    pltpu.sync_copy(vrows_v, v_hbm.at[pl.ds(base, ROWS_PER_W)])


_sc_gather = pl.kernel(
    _sc_body,
    out_type=[jax.ShapeDtypeStruct((B, EMB), jnp.float32),
              jax.ShapeDtypeStruct((B, EMB), jnp.float32)],
    mesh=plsc.VectorSubcoreMesh(core_axis_name="c", subcore_axis_name="s"),
    scratch_types=[
        pltpu.VMEM((NCHUNK_PAD, CHUNK), jnp.int32),
        pltpu.VMEM((NCHUNK, CHUNK), jnp.int32),
        pltpu.VMEM((NCHUNK, CHUNK), jnp.int32),
        pltpu.VMEM((NBUF, CHUNK, 128), jnp.float32),
        pltpu.VMEM((ROWS_PER_W, EMB), jnp.float32),
        pltpu.VMEM((ROWS_PER_W,), jnp.int32),
        pltpu.VMEM((ROWS_PER_W, 128), jnp.float32),
        pltpu.VMEM((ROWS_PER_W, EMB), jnp.float32),
        pltpu.SemaphoreType.DMA,
        pltpu.SemaphoreType.DMA,
    ],
    compiler_params=pltpu.CompilerParams(needs_layout_passes=False),
)


NFULL = 7812
NRING = 3


def _repack_body(tt_hbm, tail8_hbm, t128_hbm, ibufs, obufs, tailv,
                 semi, semo):
    wid = lax.axis_index("s") * NC + lax.axis_index("c")
    iota = lax.iota(jnp.int32, 16)

    @pl.when(wid == 0)
    def _():
        pltpu.sync_copy(tail8_hbm, tailv)
        pltpu.sync_copy(tailv, t128_hbm.at[pl.ds(NFULL * 16, 8)])

    def issue_in(k):
        bi = wid + NW * k
        pltpu.async_copy(tt_hbm.at[:, pl.ds(bi * 128, 128)],
                         ibufs.at[lax.rem(k, NRING)], semi)

    issue_in(0)
    issue_in(1)

    def step(k, c):
        bi = wid + NW * k
        slot = lax.rem(k, NRING)

        @pl.when(bi < NFULL)
        def _():
            pltpu.make_async_copy(tt_hbm.at[:, pl.ds(0, 128)],
                                  ibufs.at[0], semi).wait()

            @pl.when(k >= NRING)
            def _():
                pltpu.make_async_copy(obufs.at[0],
                                      t128_hbm.at[pl.ds(0, 16)], semo).wait()
            slotv = jnp.full((16,), slot, jnp.int32)

            def col_step(cc8, c2):
                for a in range(8):
                    cc = cc8 * 8 + a
                    row = plsc.load_gather(
                        ibufs, [slotv, iota, jnp.full((16,), cc, jnp.int32)])
                    obufs[slot, cc8, pl.ds(a * 16, 16)] = row
                return c2
            lax.fori_loop(0, 16, col_step, 0)
            pltpu.async_copy(obufs.at[slot],
                             t128_hbm.at[pl.ds(bi * 16, 16)], semo)

        @pl.when(wid + NW * (k + 2) < NFULL)
        def _():
            issue_in(k + 2)
        return c
    lax.fori_loop(0, (NFULL // NW) + 1, step, 0)

    for _ in range(NRING):
        pltpu.make_async_copy(obufs.at[0],
                              t128_hbm.at[pl.ds(0, 16)], semo).wait()


_sc_repack = pl.kernel(
    _repack_body,
    out_type=jax.ShapeDtypeStruct((TROWS, 128), jnp.float32),
    mesh=plsc.VectorSubcoreMesh(core_axis_name="c", subcore_axis_name="s"),
    scratch_types=[
        pltpu.VMEM((NRING, 16, 128), jnp.float32),
        pltpu.VMEM((NRING, 16, 128), jnp.float32),
        pltpu.VMEM((8, 128), jnp.float32),
        pltpu.SemaphoreType.DMA,
        pltpu.SemaphoreType.DMA,
    ],
    compiler_params=pltpu.CompilerParams(needs_layout_passes=False),
)


def _tc_body(u_ref, v_ref, o_ref):
    u = u_ref[...]
    v = v_ref[...]
    un = u * lax.rsqrt(jnp.maximum(jnp.sum(u * u, axis=1, keepdims=True),
                                   1e-24))
    vn = v * lax.rsqrt(jnp.maximum(jnp.sum(v * v, axis=1, keepdims=True),
                                   1e-24))
    o_ref[...] = lax.dot_general(un, vn, (((1,), (1,)), ((), ())),
                                 preferred_element_type=jnp.float32)


_tc_score = pl.pallas_call(
    _tc_body,
    out_shape=jax.ShapeDtypeStruct((B, B), jnp.float32),
)


@jax.jit
def kernel(input_seqs, items_to_predict, table):
    tail8 = table[NFULL * 128:TROWS * 8].reshape(8, 128)
    t128 = _sc_repack(table.T, tail8)
    seq_r = jnp.pad(
        input_seqs.reshape(NW, GATHER_PER_W),
        ((0, 0), (0, NCHUNK_PAD * CHUNK - GATHER_PER_W)),
    ).reshape(NW * NCHUNK_PAD, CHUNK)
    items1 = items_to_predict[:, -1].reshape(NW, ROWS_PER_W)
    u_sum, v_rows = _sc_gather(seq_r, items1, t128)
    scores = _tc_score(u_sum, v_rows)
    return scores.reshape(B, B, 1)
